# Initial kernel scaffold; baseline (speedup 1.0000x reference)
import jax
import jax.numpy as jnp
from jax.experimental import pallas as pl


def kernel(node_feat, edge_feat, edge_index, graph_ids, W_n2l, b_n2l, W_e2l, b_e2l, W_conv, b_conv, W_out, b_out):
    raise NotImplementedError("write your pallas kernel here")


# speedup vs baseline: 2.2173x; 2.2173x over previous
import functools

import jax
import jax.numpy as jnp
from jax import lax
from jax.experimental import pallas as pl
from jax.experimental.pallas import tpu as pltpu
from jax.experimental.pallas import tpu_sc as plsc

N_NODES = 10000
N_EDGES = 160000
DF = 256
DE = 16
LD = 256
OD = 128
NG = 128
HALF = 128

EB = 1000
NB = 1000
EGRID = N_EDGES // EB
NGRID = N_NODES // NB

TILES = 16
EPT = N_EDGES // TILES
IDXW = 80
RPT = EPT // IDXW
NZROWS = 1000
NZTILES = N_NODES // NZROWS

_SC_MESH = plsc.VectorSubcoreMesh(core_axis_name="c", subcore_axis_name="s")


def _tc_node_linear_body(nf_ref, w_ref, b_ref, h_ref):
    h = jnp.dot(nf_ref[...], w_ref[...], preferred_element_type=jnp.float32)
    h = h + b_ref[...]
    h_ref[0] = h[:, :HALF]
    h_ref[1] = h[:, HALF:]


def _tc_node_linear(nf, w, b):
    return pl.pallas_call(
        _tc_node_linear_body,
        grid=(NGRID,),
        in_specs=[
            pl.BlockSpec((NB, DF), lambda i: (i, 0)),
            pl.BlockSpec((DF, LD), lambda i: (0, 0)),
            pl.BlockSpec((1, LD), lambda i: (0, 0)),
        ],
        out_specs=pl.BlockSpec((2, NB, HALF), lambda i: (0, i, 0)),
        out_shape=jax.ShapeDtypeStruct((2, N_NODES, HALF), jnp.float32),
    )(nf, w, b)


def _tc_input_linear_body(ef_ref, g_ref, we_ref, be_ref, bc_ref, wc_ref,
                          m0b_ref, c1_ref):
    g = jnp.concatenate([g_ref[0], g_ref[1]], axis=1)
    m0 = jnp.dot(ef_ref[...], we_ref[...], preferred_element_type=jnp.float32)
    m0 = m0 + be_ref[...] + g
    m0b_ref[...] = m0 + bc_ref[...]
    c1 = jnp.dot(jnp.maximum(m0, 0.0), wc_ref[...],
                 preferred_element_type=jnp.float32)
    c1_ref[0] = c1[:, :HALF]
    c1_ref[1] = c1[:, HALF:]


def _tc_input_linear(ef, g0, we, be, bc, wc):
    return pl.pallas_call(
        _tc_input_linear_body,
        grid=(EGRID,),
        in_specs=[
            pl.BlockSpec((EB, DE), lambda i: (i, 0)),
            pl.BlockSpec((2, EB, HALF), lambda i: (0, i, 0)),
            pl.BlockSpec((DE, LD), lambda i: (0, 0)),
            pl.BlockSpec((1, LD), lambda i: (0, 0)),
            pl.BlockSpec((1, LD), lambda i: (0, 0)),
            pl.BlockSpec((LD, LD), lambda i: (0, 0)),
        ],
        out_specs=[
            pl.BlockSpec((EB, LD), lambda i: (i, 0)),
            pl.BlockSpec((2, EB, HALF), lambda i: (0, i, 0)),
        ],
        out_shape=[
            jax.ShapeDtypeStruct((N_EDGES, LD), jnp.float32),
            jax.ShapeDtypeStruct((2, N_EDGES, HALF), jnp.float32),
        ],
    )(ef, g0, we, be, bc, wc)


def _pair_swap(x):
    up = jnp.roll(x, -1, axis=0)
    dn = jnp.roll(x, 1, axis=0)
    par = lax.broadcasted_iota(jnp.int32, x.shape, 0) % 2
    return jnp.where(par == 0, up, dn)


def _combine(g_ref, c_ref, m0b_ref):
    g = jnp.concatenate([g_ref[0], g_ref[1]], axis=1)
    c = jnp.concatenate([c_ref[0], c_ref[1]], axis=1)
    return jnp.maximum(g - _pair_swap(c) + m0b_ref[...], 0.0)


def _tc_level_body(g_ref, c_ref, m0b_ref, wc_ref, out_ref):
    x = _combine(g_ref, c_ref, m0b_ref)
    y = jnp.dot(x, wc_ref[...], preferred_element_type=jnp.float32)
    out_ref[0] = y[:, :HALF]
    out_ref[1] = y[:, HALF:]


def _tc_level(g, c, m0b, wc):
    return pl.pallas_call(
        _tc_level_body,
        grid=(EGRID,),
        in_specs=[
            pl.BlockSpec((2, EB, HALF), lambda i: (0, i, 0)),
            pl.BlockSpec((2, EB, HALF), lambda i: (0, i, 0)),
            pl.BlockSpec((EB, LD), lambda i: (i, 0)),
            pl.BlockSpec((LD, LD), lambda i: (0, 0)),
        ],
        out_specs=pl.BlockSpec((2, EB, HALF), lambda i: (0, i, 0)),
        out_shape=jax.ShapeDtypeStruct((2, N_EDGES, HALF), jnp.float32),
    )(g, c, m0b, wc)


def _tc_last_body(g_ref, c_ref, m0b_ref, out_ref):
    x = _combine(g_ref, c_ref, m0b_ref)
    out_ref[0] = x[:, :HALF]
    out_ref[1] = x[:, HALF:]


def _tc_last(g, c, m0b):
    return pl.pallas_call(
        _tc_last_body,
        grid=(EGRID,),
        in_specs=[
            pl.BlockSpec((2, EB, HALF), lambda i: (0, i, 0)),
            pl.BlockSpec((2, EB, HALF), lambda i: (0, i, 0)),
            pl.BlockSpec((EB, LD), lambda i: (i, 0)),
        ],
        out_specs=pl.BlockSpec((2, EB, HALF), lambda i: (0, i, 0)),
        out_shape=jax.ShapeDtypeStruct((2, N_EDGES, HALF), jnp.float32),
    )(g, c, m0b)


def _tc_out_body(e2n_ref, wo_ref, bo_ref, gid_ref, y_ref, acc_ref):
    i = pl.program_id(0)

    @pl.when(i == 0)
    def _():
        acc_ref[...] = jnp.zeros_like(acc_ref)

    h = jnp.maximum(jnp.concatenate([e2n_ref[0], e2n_ref[1]], axis=1), 0.0)
    o = jnp.dot(h, wo_ref[...], preferred_element_type=jnp.float32)
    o = jnp.maximum(o + bo_ref[...], 0.0)
    gid = gid_ref[0, 0, :]
    oh = (lax.broadcasted_iota(jnp.int32, (NG, NB), 0) == gid[None, :])
    acc_ref[...] += jnp.dot(oh.astype(jnp.float32), o,
                            preferred_element_type=jnp.float32)

    @pl.when(i == pl.num_programs(0) - 1)
    def _():
        y_ref[...] = jnp.maximum(acc_ref[...], 0.0)


def _tc_out(e2n, wo, bo, gid3):
    return pl.pallas_call(
        _tc_out_body,
        grid=(NGRID,),
        in_specs=[
            pl.BlockSpec((2, NB, HALF), lambda i: (0, i, 0)),
            pl.BlockSpec((LD, OD), lambda i: (0, 0)),
            pl.BlockSpec((1, OD), lambda i: (0, 0)),
            pl.BlockSpec((1, 1, NB), lambda i: (i, 0, 0)),
        ],
        out_specs=pl.BlockSpec((NG, OD), lambda i: (0, 0)),
        out_shape=jax.ShapeDtypeStruct((NG, OD), jnp.float32),
        scratch_shapes=[pltpu.VMEM((NG, OD), jnp.float32)],
    )(e2n, wo, bo, gid3)


def _sc_gather_rows(table_ref, out_ref, c, s, idxbuf, rowbuf, sem):
    def row(j, carry):
        e0 = s * EPT + j * IDXW
        pltpu.async_copy(table_ref.at[idxbuf.at[j]], rowbuf, sem).wait()
        pltpu.sync_copy(rowbuf, out_ref.at[c, pl.ds(e0, IDXW)])
        return carry

    lax.fori_loop(0, RPT, row, 0)


@functools.partial(
    pl.kernel, mesh=_SC_MESH,
    out_type=jax.ShapeDtypeStruct((2, N_EDGES, HALF), jnp.float32),
    scratch_types=[
        pltpu.VMEM((RPT, IDXW), jnp.int32),
        pltpu.VMEM((IDXW, HALF), jnp.float32),
        pltpu.SemaphoreType.DMA,
    ],
)
def _sc_gather(h0, h1, idx3, out, idxbuf, rowbuf, sem):
    c = lax.axis_index("c")
    s = lax.axis_index("s")
    pltpu.sync_copy(idx3.at[s], idxbuf)

    @pl.when(c == 0)
    def _():
        _sc_gather_rows(h0, out, c, s, idxbuf, rowbuf, sem)

    @pl.when(c == 1)
    def _():
        _sc_gather_rows(h1, out, c, s, idxbuf, rowbuf, sem)


def _sc_zero_acc(zer_ref, acc, s):
    @pl.when(s < NZTILES)
    def _():
        pltpu.sync_copy(zer_ref, acc.at[pl.ds(s * NZROWS, NZROWS)])


def _sc_scatter_add(cmat_ref, acc, c, s, idxbuf, rowbuf):
    def row(j, carry):
        e0 = s * EPT + j * IDXW
        pltpu.sync_copy(cmat_ref.at[c, pl.ds(e0, IDXW)], rowbuf)
        pltpu.sync_copy(rowbuf, acc.at[idxbuf.at[j]], add=True)
        return carry

    lax.fori_loop(0, RPT, row, 0)


_LEVEL_SCRATCH = [
    pltpu.VMEM_SHARED((N_NODES, HALF), jnp.float32),
    pltpu.VMEM((RPT, IDXW), jnp.int32),
    pltpu.VMEM((IDXW, HALF), jnp.float32),
    pltpu.SemaphoreType.DMA,
]


@functools.partial(
    pl.kernel, mesh=_SC_MESH,
    out_type=jax.ShapeDtypeStruct((2, N_EDGES, HALF), jnp.float32),
    scratch_types=_LEVEL_SCRATCH,
)
def _sc_level(cmat, src3, dst3, zer, g_out, acc, idxbuf, rowbuf, sem):
    c = lax.axis_index("c")
    s = lax.axis_index("s")
    pltpu.sync_copy(dst3.at[s], idxbuf)
    _sc_zero_acc(zer, acc, s)
    plsc.subcore_barrier()
    _sc_scatter_add(cmat, acc, c, s, idxbuf, rowbuf)
    plsc.subcore_barrier()
    pltpu.sync_copy(src3.at[s], idxbuf)

    def row(j, carry):
        e0 = s * EPT + j * IDXW
        pltpu.async_copy(acc.at[idxbuf.at[j]], rowbuf, sem).wait()
        pltpu.sync_copy(rowbuf, g_out.at[c, pl.ds(e0, IDXW)])
        return carry

    lax.fori_loop(0, RPT, row, 0)


@functools.partial(
    pl.kernel, mesh=_SC_MESH,
    out_type=jax.ShapeDtypeStruct((2, N_NODES, HALF), jnp.float32),
    scratch_types=_LEVEL_SCRATCH,
)
def _sc_scatter(cmat, dst3, zer, out, acc, idxbuf, rowbuf, sem):
    c = lax.axis_index("c")
    s = lax.axis_index("s")
    pltpu.sync_copy(dst3.at[s], idxbuf)
    _sc_zero_acc(zer, acc, s)
    plsc.subcore_barrier()
    _sc_scatter_add(cmat, acc, c, s, idxbuf, rowbuf)
    plsc.subcore_barrier()

    @pl.when(s < NZTILES)
    def _():
        pltpu.sync_copy(acc.at[pl.ds(s * NZROWS, NZROWS)],
                        out.at[c, pl.ds(s * NZROWS, NZROWS)])


def kernel(node_feat, edge_feat, edge_index, graph_ids, W_n2l, b_n2l,
           W_e2l, b_e2l, W_conv, b_conv, W_out, b_out):
    src3 = edge_index[0].reshape(TILES, RPT, IDXW)
    dst3 = edge_index[1].reshape(TILES, RPT, IDXW)
    gid3 = graph_ids.reshape(NGRID, 1, NB)
    zer = jnp.zeros((NZROWS, HALF), jnp.float32)
    bn = b_n2l.reshape(1, LD)
    be = b_e2l.reshape(1, LD)
    bc = b_conv.reshape(1, LD)
    bo = b_out.reshape(1, OD)

    h = _tc_node_linear(node_feat, W_n2l, bn)
    g0 = _sc_gather(h[0], h[1], src3)
    m0b, c = _tc_input_linear(edge_feat, g0, W_e2l, be, bc, W_conv)
    for _ in range(2):
        g = _sc_level(c, src3, dst3, zer)
        c = _tc_level(g, c, m0b, W_conv)
    g = _sc_level(c, src3, dst3, zer)
    cur = _tc_last(g, c, m0b)
    e2n = _sc_scatter(cur, dst3, zer)
    return _tc_out(e2n, W_out, bo, gid3)


# speedup vs baseline: 2.5601x; 1.1546x over previous
import functools

import jax
import jax.numpy as jnp
from jax import lax
from jax.experimental import pallas as pl
from jax.experimental.pallas import tpu as pltpu
from jax.experimental.pallas import tpu_sc as plsc

N_NODES = 10000
N_EDGES = 160000
DF = 256
DE = 16
LD = 256
OD = 128
NG = 128
HALF = 128

EB = 1000
NB = 1000
EGRID = N_EDGES // EB
NGRID = N_NODES // NB

TILES = 16
EPT = N_EDGES // TILES
IDXW = 80
RPT = EPT // IDXW
NZROWS = 1000
NZTILES = N_NODES // NZROWS

_SC_MESH = plsc.VectorSubcoreMesh(core_axis_name="c", subcore_axis_name="s")


def _tc_node_linear_body(nf_ref, w_ref, b_ref, h_ref):
    h = jnp.dot(nf_ref[...], w_ref[...], preferred_element_type=jnp.float32)
    h = h + b_ref[...]
    h_ref[0] = h[:, :HALF]
    h_ref[1] = h[:, HALF:]


def _tc_node_linear(nf, w, b):
    return pl.pallas_call(
        _tc_node_linear_body,
        grid=(NGRID,),
        in_specs=[
            pl.BlockSpec((NB, DF), lambda i: (i, 0)),
            pl.BlockSpec((DF, LD), lambda i: (0, 0)),
            pl.BlockSpec((1, LD), lambda i: (0, 0)),
        ],
        out_specs=pl.BlockSpec((2, NB, HALF), lambda i: (0, i, 0)),
        out_shape=jax.ShapeDtypeStruct((2, N_NODES, HALF), jnp.float32),
    )(nf, w, b)


def _tc_input_linear_body(ef_ref, g_ref, we_ref, be_ref, bc_ref, wc_ref,
                          m0b_ref, c1_ref):
    g = jnp.concatenate([g_ref[0], g_ref[1]], axis=1)
    m0 = jnp.dot(ef_ref[...], we_ref[...], preferred_element_type=jnp.float32)
    m0 = m0 + be_ref[...] + g
    m0b_ref[...] = m0 + bc_ref[...]
    c1 = jnp.dot(jnp.maximum(m0, 0.0), wc_ref[...],
                 preferred_element_type=jnp.float32)
    c1_ref[0] = c1[:, :HALF]
    c1_ref[1] = c1[:, HALF:]


def _tc_input_linear(ef, g0, we, be, bc, wc):
    return pl.pallas_call(
        _tc_input_linear_body,
        grid=(EGRID,),
        in_specs=[
            pl.BlockSpec((EB, DE), lambda i: (i, 0)),
            pl.BlockSpec((2, EB, HALF), lambda i: (0, i, 0)),
            pl.BlockSpec((DE, LD), lambda i: (0, 0)),
            pl.BlockSpec((1, LD), lambda i: (0, 0)),
            pl.BlockSpec((1, LD), lambda i: (0, 0)),
            pl.BlockSpec((LD, LD), lambda i: (0, 0)),
        ],
        out_specs=[
            pl.BlockSpec((EB, LD), lambda i: (i, 0)),
            pl.BlockSpec((2, EB, HALF), lambda i: (0, i, 0)),
        ],
        out_shape=[
            jax.ShapeDtypeStruct((N_EDGES, LD), jnp.float32),
            jax.ShapeDtypeStruct((2, N_EDGES, HALF), jnp.float32),
        ],
    )(ef, g0, we, be, bc, wc)


def _pair_swap(x):
    up = jnp.roll(x, -1, axis=0)
    dn = jnp.roll(x, 1, axis=0)
    par = lax.broadcasted_iota(jnp.int32, x.shape, 0) % 2
    return jnp.where(par == 0, up, dn)


def _combine(g_ref, c_ref, m0b_ref):
    g = jnp.concatenate([g_ref[0], g_ref[1]], axis=1)
    c = jnp.concatenate([c_ref[0], c_ref[1]], axis=1)
    return jnp.maximum(g - _pair_swap(c) + m0b_ref[...], 0.0)


def _tc_level_body(g_ref, c_ref, m0b_ref, wc_ref, out_ref):
    x = _combine(g_ref, c_ref, m0b_ref)
    y = jnp.dot(x, wc_ref[...], preferred_element_type=jnp.float32)
    out_ref[0] = y[:, :HALF]
    out_ref[1] = y[:, HALF:]


def _tc_level(g, c, m0b, wc):
    return pl.pallas_call(
        _tc_level_body,
        grid=(EGRID,),
        in_specs=[
            pl.BlockSpec((2, EB, HALF), lambda i: (0, i, 0)),
            pl.BlockSpec((2, EB, HALF), lambda i: (0, i, 0)),
            pl.BlockSpec((EB, LD), lambda i: (i, 0)),
            pl.BlockSpec((LD, LD), lambda i: (0, 0)),
        ],
        out_specs=pl.BlockSpec((2, EB, HALF), lambda i: (0, i, 0)),
        out_shape=jax.ShapeDtypeStruct((2, N_EDGES, HALF), jnp.float32),
    )(g, c, m0b, wc)


def _tc_last_body(g_ref, c_ref, m0b_ref, out_ref):
    x = _combine(g_ref, c_ref, m0b_ref)
    out_ref[0] = x[:, :HALF]
    out_ref[1] = x[:, HALF:]


def _tc_last(g, c, m0b):
    return pl.pallas_call(
        _tc_last_body,
        grid=(EGRID,),
        in_specs=[
            pl.BlockSpec((2, EB, HALF), lambda i: (0, i, 0)),
            pl.BlockSpec((2, EB, HALF), lambda i: (0, i, 0)),
            pl.BlockSpec((EB, LD), lambda i: (i, 0)),
        ],
        out_specs=pl.BlockSpec((2, EB, HALF), lambda i: (0, i, 0)),
        out_shape=jax.ShapeDtypeStruct((2, N_EDGES, HALF), jnp.float32),
    )(g, c, m0b)


def _tc_out_body(e2n_ref, wo_ref, bo_ref, gid_ref, y_ref, acc_ref):
    i = pl.program_id(0)

    @pl.when(i == 0)
    def _():
        acc_ref[...] = jnp.zeros_like(acc_ref)

    h = jnp.maximum(jnp.concatenate([e2n_ref[0], e2n_ref[1]], axis=1), 0.0)
    o = jnp.dot(h, wo_ref[...], preferred_element_type=jnp.float32)
    o = jnp.maximum(o + bo_ref[...], 0.0)
    gid = gid_ref[0, 0, :]
    oh = (lax.broadcasted_iota(jnp.int32, (NG, NB), 0) == gid[None, :])
    acc_ref[...] += jnp.dot(oh.astype(jnp.float32), o,
                            preferred_element_type=jnp.float32)

    @pl.when(i == pl.num_programs(0) - 1)
    def _():
        y_ref[...] = jnp.maximum(acc_ref[...], 0.0)


def _tc_out(e2n, wo, bo, gid3):
    return pl.pallas_call(
        _tc_out_body,
        grid=(NGRID,),
        in_specs=[
            pl.BlockSpec((2, NB, HALF), lambda i: (0, i, 0)),
            pl.BlockSpec((LD, OD), lambda i: (0, 0)),
            pl.BlockSpec((1, OD), lambda i: (0, 0)),
            pl.BlockSpec((1, 1, NB), lambda i: (i, 0, 0)),
        ],
        out_specs=pl.BlockSpec((NG, OD), lambda i: (0, 0)),
        out_shape=jax.ShapeDtypeStruct((NG, OD), jnp.float32),
        scratch_shapes=[pltpu.VMEM((NG, OD), jnp.float32)],
    )(e2n, wo, bo, gid3)


NBUF = 3
NROUND = RPT // NBUF
TAIL = NROUND * NBUF


def _sc_gather_rows(table_ref, out_ref, c, s, idxbuf, rowbuf, sem):
    def one(j, b):
        e0 = s * EPT + j * IDXW
        pltpu.async_copy(table_ref.at[idxbuf.at[j]], rowbuf.at[b], sem).wait()
        pltpu.sync_copy(rowbuf.at[b], out_ref.at[c, pl.ds(e0, IDXW)])

    def rnd(r, carry):
        hs = [pltpu.async_copy(table_ref.at[idxbuf.at[r * NBUF + b]],
                               rowbuf.at[b], sem) for b in range(NBUF)]
        for h in hs:
            h.wait()
        hs = [pltpu.async_copy(
                  rowbuf.at[b],
                  out_ref.at[c, pl.ds(s * EPT + (r * NBUF + b) * IDXW, IDXW)],
                  sem) for b in range(NBUF)]
        for h in hs:
            h.wait()
        return carry

    lax.fori_loop(0, NROUND, rnd, 0)
    for t in range(TAIL, RPT):
        one(t, t - TAIL)


_SC_SCRATCH = [
    pltpu.VMEM((RPT, IDXW), jnp.int32),
    pltpu.VMEM((NBUF, IDXW, HALF), jnp.float32),
    pltpu.SemaphoreType.DMA,
]


@functools.partial(
    pl.kernel, mesh=_SC_MESH,
    out_type=jax.ShapeDtypeStruct((2, N_EDGES, HALF), jnp.float32),
    scratch_types=_SC_SCRATCH,
)
def _sc_gather(h0, h1, idx3, out, idxbuf, rowbuf, sem):
    c = lax.axis_index("c")
    s = lax.axis_index("s")
    pltpu.sync_copy(idx3.at[s], idxbuf)

    @pl.when(c == 0)
    def _():
        _sc_gather_rows(h0, out, c, s, idxbuf, rowbuf, sem)

    @pl.when(c == 1)
    def _():
        _sc_gather_rows(h1, out, c, s, idxbuf, rowbuf, sem)


def _sc_zero_acc(zer_ref, acc, s):
    @pl.when(s < NZTILES)
    def _():
        pltpu.sync_copy(zer_ref, acc.at[pl.ds(s * NZROWS, NZROWS)])


def _sc_scatter_add(cmat_ref, acc, c, s, idxbuf, rowbuf, sem):
    def rnd(r, carry):
        hs = [pltpu.async_copy(
                  cmat_ref.at[c, pl.ds(s * EPT + (r * NBUF + b) * IDXW, IDXW)],
                  rowbuf.at[b], sem) for b in range(NBUF)]
        for h in hs:
            h.wait()
        hs = [pltpu.async_copy(rowbuf.at[b], acc.at[idxbuf.at[r * NBUF + b]],
                               sem, add=True) for b in range(NBUF)]
        for h in hs:
            h.wait()
        return carry

    lax.fori_loop(0, NROUND, rnd, 0)
    for t in range(TAIL, RPT):
        e0 = s * EPT + t * IDXW
        pltpu.sync_copy(cmat_ref.at[c, pl.ds(e0, IDXW)], rowbuf.at[t - TAIL])
        pltpu.sync_copy(rowbuf.at[t - TAIL], acc.at[idxbuf.at[t]], add=True)


_LEVEL_SCRATCH = [
    pltpu.VMEM_SHARED((N_NODES, HALF), jnp.float32),
    pltpu.VMEM((RPT, IDXW), jnp.int32),
    pltpu.VMEM((NBUF, IDXW, HALF), jnp.float32),
    pltpu.SemaphoreType.DMA,
]


@functools.partial(
    pl.kernel, mesh=_SC_MESH,
    out_type=jax.ShapeDtypeStruct((2, N_EDGES, HALF), jnp.float32),
    scratch_types=_LEVEL_SCRATCH,
)
def _sc_level(cmat, src3, dst3, zer, g_out, acc, idxbuf, rowbuf, sem):
    c = lax.axis_index("c")
    s = lax.axis_index("s")
    pltpu.sync_copy(dst3.at[s], idxbuf)
    _sc_zero_acc(zer, acc, s)
    plsc.subcore_barrier()
    _sc_scatter_add(cmat, acc, c, s, idxbuf, rowbuf, sem)
    plsc.subcore_barrier()
    pltpu.sync_copy(src3.at[s], idxbuf)
    _sc_gather_rows(acc, g_out, c, s, idxbuf, rowbuf, sem)


@functools.partial(
    pl.kernel, mesh=_SC_MESH,
    out_type=jax.ShapeDtypeStruct((2, N_NODES, HALF), jnp.float32),
    scratch_types=_LEVEL_SCRATCH,
)
def _sc_scatter(cmat, dst3, zer, out, acc, idxbuf, rowbuf, sem):
    c = lax.axis_index("c")
    s = lax.axis_index("s")
    pltpu.sync_copy(dst3.at[s], idxbuf)
    _sc_zero_acc(zer, acc, s)
    plsc.subcore_barrier()
    _sc_scatter_add(cmat, acc, c, s, idxbuf, rowbuf, sem)
    plsc.subcore_barrier()

    @pl.when(s < NZTILES)
    def _():
        pltpu.sync_copy(acc.at[pl.ds(s * NZROWS, NZROWS)],
                        out.at[c, pl.ds(s * NZROWS, NZROWS)])


def kernel(node_feat, edge_feat, edge_index, graph_ids, W_n2l, b_n2l,
           W_e2l, b_e2l, W_conv, b_conv, W_out, b_out):
    src3 = edge_index[0].reshape(TILES, RPT, IDXW)
    dst3 = edge_index[1].reshape(TILES, RPT, IDXW)
    gid3 = graph_ids.reshape(NGRID, 1, NB)
    zer = jnp.zeros((NZROWS, HALF), jnp.float32)
    bn = b_n2l.reshape(1, LD)
    be = b_e2l.reshape(1, LD)
    bc = b_conv.reshape(1, LD)
    bo = b_out.reshape(1, OD)

    h = _tc_node_linear(node_feat, W_n2l, bn)
    g0 = _sc_gather(h[0], h[1], src3)
    m0b, c = _tc_input_linear(edge_feat, g0, W_e2l, be, bc, W_conv)
    for _ in range(2):
        g = _sc_level(c, src3, dst3, zer)
        c = _tc_level(g, c, m0b, W_conv)
    g = _sc_level(c, src3, dst3, zer)
    cur = _tc_last(g, c, m0b)
    e2n = _sc_scatter(cur, dst3, zer)
    return _tc_out(e2n, W_out, bo, gid3)


# speedup vs baseline: 2.5610x; 1.0003x over previous
import functools

import jax
import jax.numpy as jnp
from jax import lax
from jax.experimental import pallas as pl
from jax.experimental.pallas import tpu as pltpu
from jax.experimental.pallas import tpu_sc as plsc

N_NODES = 10000
N_EDGES = 160000
DF = 256
DE = 16
LD = 256
OD = 128
NG = 128
HALF = 128

EB = 1000
NB = 1000
EGRID = N_EDGES // EB
NGRID = N_NODES // NB

TILES = 16
EPT = N_EDGES // TILES
IDXW = 80
RPT = EPT // IDXW
NZROWS = 1000
NZTILES = N_NODES // NZROWS

_SC_MESH = plsc.VectorSubcoreMesh(core_axis_name="c", subcore_axis_name="s")


def _tc_node_linear_body(nf_ref, w_ref, b_ref, h_ref):
    h = jnp.dot(nf_ref[...], w_ref[...], preferred_element_type=jnp.float32)
    h = h + b_ref[...]
    h_ref[0] = h[:, :HALF]
    h_ref[1] = h[:, HALF:]


def _tc_node_linear(nf, w, b):
    return pl.pallas_call(
        _tc_node_linear_body,
        grid=(NGRID,),
        in_specs=[
            pl.BlockSpec((NB, DF), lambda i: (i, 0)),
            pl.BlockSpec((DF, LD), lambda i: (0, 0)),
            pl.BlockSpec((1, LD), lambda i: (0, 0)),
        ],
        out_specs=pl.BlockSpec((2, NB, HALF), lambda i: (0, i, 0)),
        out_shape=jax.ShapeDtypeStruct((2, N_NODES, HALF), jnp.float32),
    )(nf, w, b)


def _tc_input_linear_body(ef_ref, g_ref, we_ref, be_ref, bc_ref, wc_ref,
                          m0b_ref, c1_ref):
    g = jnp.concatenate([g_ref[0], g_ref[1]], axis=1)
    m0 = jnp.dot(ef_ref[...], we_ref[...], preferred_element_type=jnp.float32)
    m0 = m0 + be_ref[...] + g
    m0b_ref[...] = m0 + bc_ref[...]
    c1 = jnp.dot(jnp.maximum(m0, 0.0), wc_ref[...],
                 preferred_element_type=jnp.float32)
    c1_ref[0] = c1[:, :HALF]
    c1_ref[1] = c1[:, HALF:]


def _tc_input_linear(ef, g0, we, be, bc, wc):
    return pl.pallas_call(
        _tc_input_linear_body,
        grid=(EGRID,),
        in_specs=[
            pl.BlockSpec((EB, DE), lambda i: (i, 0)),
            pl.BlockSpec((2, EB, HALF), lambda i: (0, i, 0)),
            pl.BlockSpec((DE, LD), lambda i: (0, 0)),
            pl.BlockSpec((1, LD), lambda i: (0, 0)),
            pl.BlockSpec((1, LD), lambda i: (0, 0)),
            pl.BlockSpec((LD, LD), lambda i: (0, 0)),
        ],
        out_specs=[
            pl.BlockSpec((EB, LD), lambda i: (i, 0)),
            pl.BlockSpec((2, EB, HALF), lambda i: (0, i, 0)),
        ],
        out_shape=[
            jax.ShapeDtypeStruct((N_EDGES, LD), jnp.float32),
            jax.ShapeDtypeStruct((2, N_EDGES, HALF), jnp.float32),
        ],
    )(ef, g0, we, be, bc, wc)


def _pair_swap(x):
    up = jnp.roll(x, -1, axis=0)
    dn = jnp.roll(x, 1, axis=0)
    par = lax.broadcasted_iota(jnp.int32, x.shape, 0) % 2
    return jnp.where(par == 0, up, dn)


def _combine(g_ref, c_ref, m0b_ref):
    g = jnp.concatenate([g_ref[0], g_ref[1]], axis=1)
    c = jnp.concatenate([c_ref[0], c_ref[1]], axis=1)
    return jnp.maximum(g - _pair_swap(c) + m0b_ref[...], 0.0)


def _tc_level_body(g_ref, c_ref, m0b_ref, wc_ref, out_ref):
    x = _combine(g_ref, c_ref, m0b_ref)
    y = jnp.dot(x, wc_ref[...], preferred_element_type=jnp.float32)
    out_ref[0] = y[:, :HALF]
    out_ref[1] = y[:, HALF:]


def _tc_level(g, c, m0b, wc):
    return pl.pallas_call(
        _tc_level_body,
        grid=(EGRID,),
        in_specs=[
            pl.BlockSpec((2, EB, HALF), lambda i: (0, i, 0)),
            pl.BlockSpec((2, EB, HALF), lambda i: (0, i, 0)),
            pl.BlockSpec((EB, LD), lambda i: (i, 0)),
            pl.BlockSpec((LD, LD), lambda i: (0, 0)),
        ],
        out_specs=pl.BlockSpec((2, EB, HALF), lambda i: (0, i, 0)),
        out_shape=jax.ShapeDtypeStruct((2, N_EDGES, HALF), jnp.float32),
    )(g, c, m0b, wc)


def _tc_last_body(g_ref, c_ref, m0b_ref, out_ref):
    x = _combine(g_ref, c_ref, m0b_ref)
    out_ref[0] = x[:, :HALF]
    out_ref[1] = x[:, HALF:]


def _tc_last(g, c, m0b):
    return pl.pallas_call(
        _tc_last_body,
        grid=(EGRID,),
        in_specs=[
            pl.BlockSpec((2, EB, HALF), lambda i: (0, i, 0)),
            pl.BlockSpec((2, EB, HALF), lambda i: (0, i, 0)),
            pl.BlockSpec((EB, LD), lambda i: (i, 0)),
        ],
        out_specs=pl.BlockSpec((2, EB, HALF), lambda i: (0, i, 0)),
        out_shape=jax.ShapeDtypeStruct((2, N_EDGES, HALF), jnp.float32),
    )(g, c, m0b)


def _tc_out_body(e2n_ref, wo_ref, bo_ref, gid_ref, y_ref, acc_ref):
    i = pl.program_id(0)

    @pl.when(i == 0)
    def _():
        acc_ref[...] = jnp.zeros_like(acc_ref)

    h = jnp.maximum(jnp.concatenate([e2n_ref[0], e2n_ref[1]], axis=1), 0.0)
    o = jnp.dot(h, wo_ref[...], preferred_element_type=jnp.float32)
    o = jnp.maximum(o + bo_ref[...], 0.0)
    gid = gid_ref[0, 0, :]
    oh = (lax.broadcasted_iota(jnp.int32, (NG, NB), 0) == gid[None, :])
    acc_ref[...] += jnp.dot(oh.astype(jnp.float32), o,
                            preferred_element_type=jnp.float32)

    @pl.when(i == pl.num_programs(0) - 1)
    def _():
        y_ref[...] = jnp.maximum(acc_ref[...], 0.0)


def _tc_out(e2n, wo, bo, gid3):
    return pl.pallas_call(
        _tc_out_body,
        grid=(NGRID,),
        in_specs=[
            pl.BlockSpec((2, NB, HALF), lambda i: (0, i, 0)),
            pl.BlockSpec((LD, OD), lambda i: (0, 0)),
            pl.BlockSpec((1, OD), lambda i: (0, 0)),
            pl.BlockSpec((1, 1, NB), lambda i: (i, 0, 0)),
        ],
        out_specs=pl.BlockSpec((NG, OD), lambda i: (0, 0)),
        out_shape=jax.ShapeDtypeStruct((NG, OD), jnp.float32),
        scratch_shapes=[pltpu.VMEM((NG, OD), jnp.float32)],
    )(e2n, wo, bo, gid3)


NBUF = 3
NROUND = RPT // NBUF
TAIL = NROUND * NBUF


def _sc_gather_rows(table_ref, out_ref, c, s, idxbuf, rowbuf, gsem, ssem):
    def fire_gather(j, b):
        pltpu.async_copy(table_ref.at[idxbuf.at[j]], rowbuf.at[b], gsem)

    for b in range(NBUF):
        fire_gather(b, b)

    def rnd(r, carry):
        for b in range(NBUF):
            pltpu.make_async_copy(table_ref.at[idxbuf.at[r * NBUF + b]],
                                  rowbuf.at[b], gsem).wait()
        hs = [pltpu.async_copy(
                  rowbuf.at[b],
                  out_ref.at[c, pl.ds(s * EPT + (r * NBUF + b) * IDXW, IDXW)],
                  ssem) for b in range(NBUF)]
        for h in hs:
            h.wait()

        @pl.when(r + 1 < NROUND)
        def _():
            for b in range(NBUF):
                fire_gather((r + 1) * NBUF + b, b)

        return carry

    lax.fori_loop(0, NROUND, rnd, 0)
    for t in range(TAIL, RPT):
        b = t - TAIL
        pltpu.async_copy(table_ref.at[idxbuf.at[t]], rowbuf.at[b], gsem).wait()
        pltpu.sync_copy(rowbuf.at[b],
                        out_ref.at[c, pl.ds(s * EPT + t * IDXW, IDXW)])


_SC_SCRATCH = [
    pltpu.VMEM((RPT, IDXW), jnp.int32),
    pltpu.VMEM((NBUF, IDXW, HALF), jnp.float32),
    pltpu.SemaphoreType.DMA,
    pltpu.SemaphoreType.DMA,
]


@functools.partial(
    pl.kernel, mesh=_SC_MESH,
    out_type=jax.ShapeDtypeStruct((2, N_EDGES, HALF), jnp.float32),
    scratch_types=_SC_SCRATCH,
)
def _sc_gather(h0, h1, idx3, out, idxbuf, rowbuf, sem, sem2):
    c = lax.axis_index("c")
    s = lax.axis_index("s")
    pltpu.sync_copy(idx3.at[s], idxbuf)

    @pl.when(c == 0)
    def _():
        _sc_gather_rows(h0, out, c, s, idxbuf, rowbuf, sem, sem2)

    @pl.when(c == 1)
    def _():
        _sc_gather_rows(h1, out, c, s, idxbuf, rowbuf, sem, sem2)


def _sc_zero_acc(zer_ref, acc, s):
    @pl.when(s < NZTILES)
    def _():
        pltpu.sync_copy(zer_ref, acc.at[pl.ds(s * NZROWS, NZROWS)])


def _sc_scatter_add(cmat_ref, acc, c, s, idxbuf, rowbuf, lsem, asem):
    def src_slice(j):
        return cmat_ref.at[c, pl.ds(s * EPT + j * IDXW, IDXW)]

    for b in range(NBUF):
        pltpu.async_copy(src_slice(b), rowbuf.at[b], lsem)

    def rnd(r, carry):
        for b in range(NBUF):
            pltpu.make_async_copy(src_slice(r * NBUF + b), rowbuf.at[b],
                                  lsem).wait()
        hs = [pltpu.async_copy(rowbuf.at[b], acc.at[idxbuf.at[r * NBUF + b]],
                               asem, add=True) for b in range(NBUF)]
        for h in hs:
            h.wait()

        @pl.when(r + 1 < NROUND)
        def _():
            for b in range(NBUF):
                pltpu.async_copy(src_slice((r + 1) * NBUF + b), rowbuf.at[b],
                                 lsem)

        return carry

    lax.fori_loop(0, NROUND, rnd, 0)
    for t in range(TAIL, RPT):
        pltpu.sync_copy(src_slice(t), rowbuf.at[t - TAIL])
        pltpu.sync_copy(rowbuf.at[t - TAIL], acc.at[idxbuf.at[t]], add=True)


_LEVEL_SCRATCH = [
    pltpu.VMEM_SHARED((N_NODES, HALF), jnp.float32),
    pltpu.VMEM((RPT, IDXW), jnp.int32),
    pltpu.VMEM((NBUF, IDXW, HALF), jnp.float32),
    pltpu.SemaphoreType.DMA,
    pltpu.SemaphoreType.DMA,
]


@functools.partial(
    pl.kernel, mesh=_SC_MESH,
    out_type=jax.ShapeDtypeStruct((2, N_EDGES, HALF), jnp.float32),
    scratch_types=_LEVEL_SCRATCH,
)
def _sc_level(cmat, src3, dst3, zer, g_out, acc, idxbuf, rowbuf, sem, sem2):
    c = lax.axis_index("c")
    s = lax.axis_index("s")
    pltpu.sync_copy(dst3.at[s], idxbuf)
    _sc_zero_acc(zer, acc, s)
    plsc.subcore_barrier()
    _sc_scatter_add(cmat, acc, c, s, idxbuf, rowbuf, sem, sem2)
    plsc.subcore_barrier()
    pltpu.sync_copy(src3.at[s], idxbuf)
    _sc_gather_rows(acc, g_out, c, s, idxbuf, rowbuf, sem, sem2)


@functools.partial(
    pl.kernel, mesh=_SC_MESH,
    out_type=jax.ShapeDtypeStruct((2, N_NODES, HALF), jnp.float32),
    scratch_types=_LEVEL_SCRATCH,
)
def _sc_scatter(cmat, dst3, zer, out, acc, idxbuf, rowbuf, sem, sem2):
    c = lax.axis_index("c")
    s = lax.axis_index("s")
    pltpu.sync_copy(dst3.at[s], idxbuf)
    _sc_zero_acc(zer, acc, s)
    plsc.subcore_barrier()
    _sc_scatter_add(cmat, acc, c, s, idxbuf, rowbuf, sem, sem2)
    plsc.subcore_barrier()

    @pl.when(s < NZTILES)
    def _():
        pltpu.sync_copy(acc.at[pl.ds(s * NZROWS, NZROWS)],
                        out.at[c, pl.ds(s * NZROWS, NZROWS)])


def kernel(node_feat, edge_feat, edge_index, graph_ids, W_n2l, b_n2l,
           W_e2l, b_e2l, W_conv, b_conv, W_out, b_out):
    src3 = edge_index[0].reshape(TILES, RPT, IDXW)
    dst3 = edge_index[1].reshape(TILES, RPT, IDXW)
    gid3 = graph_ids.reshape(NGRID, 1, NB)
    zer = jnp.zeros((NZROWS, HALF), jnp.float32)
    bn = b_n2l.reshape(1, LD)
    be = b_e2l.reshape(1, LD)
    bc = b_conv.reshape(1, LD)
    bo = b_out.reshape(1, OD)

    h = _tc_node_linear(node_feat, W_n2l, bn)
    g0 = _sc_gather(h[0], h[1], src3)
    m0b, c = _tc_input_linear(edge_feat, g0, W_e2l, be, bc, W_conv)
    for _ in range(2):
        g = _sc_level(c, src3, dst3, zer)
        c = _tc_level(g, c, m0b, W_conv)
    g = _sc_level(c, src3, dst3, zer)
    cur = _tc_last(g, c, m0b)
    e2n = _sc_scatter(cur, dst3, zer)
    return _tc_out(e2n, W_out, bo, gid3)


# speedup vs baseline: 2.6246x; 1.0248x over previous
import functools

import jax
import jax.numpy as jnp
from jax import lax
from jax.experimental import pallas as pl
from jax.experimental.pallas import tpu as pltpu
from jax.experimental.pallas import tpu_sc as plsc

N_NODES = 10000
N_EDGES = 160000
DF = 256
DE = 16
LD = 256
OD = 128
NG = 128
HALF = 128

EB = 1000
NB = 1000
EGRID = N_EDGES // EB
NGRID = N_NODES // NB

TILES = 16
EPT = N_EDGES // TILES
IDXW = 80
RPT = EPT // IDXW
NZROWS = 1000
NZTILES = N_NODES // NZROWS

_SC_MESH = plsc.VectorSubcoreMesh(core_axis_name="c", subcore_axis_name="s")


def _tc_node_linear_body(nf_ref, w_ref, b_ref, h_ref):
    h = jnp.dot(nf_ref[...], w_ref[...], preferred_element_type=jnp.float32)
    h = h + b_ref[...]
    h_ref[0] = h[:, :HALF]
    h_ref[1] = h[:, HALF:]


def _tc_node_linear(nf, w, b):
    return pl.pallas_call(
        _tc_node_linear_body,
        grid=(NGRID,),
        in_specs=[
            pl.BlockSpec((NB, DF), lambda i: (i, 0)),
            pl.BlockSpec((DF, LD), lambda i: (0, 0)),
            pl.BlockSpec((1, LD), lambda i: (0, 0)),
        ],
        out_specs=pl.BlockSpec((2, NB, HALF), lambda i: (0, i, 0)),
        out_shape=jax.ShapeDtypeStruct((2, N_NODES, HALF), jnp.float32),
    )(nf, w, b)


def _tc_input_linear_body(ef_ref, g_ref, we_ref, be_ref, bc_ref, wc_ref,
                          m0b_ref, c1_ref):
    g = jnp.concatenate([g_ref[0], g_ref[1]], axis=1)
    m0 = jnp.dot(ef_ref[...], we_ref[...], preferred_element_type=jnp.float32)
    m0 = m0 + be_ref[...] + g
    m0b_ref[...] = (m0 + bc_ref[...]).astype(jnp.bfloat16)
    c1 = jnp.dot(jnp.maximum(m0, 0.0), wc_ref[...],
                 preferred_element_type=jnp.float32)
    c1_ref[0] = c1[:, :HALF]
    c1_ref[1] = c1[:, HALF:]


def _tc_input_linear(ef, g0, we, be, bc, wc):
    return pl.pallas_call(
        _tc_input_linear_body,
        grid=(EGRID,),
        in_specs=[
            pl.BlockSpec((EB, DE), lambda i: (i, 0)),
            pl.BlockSpec((2, EB, HALF), lambda i: (0, i, 0)),
            pl.BlockSpec((DE, LD), lambda i: (0, 0)),
            pl.BlockSpec((1, LD), lambda i: (0, 0)),
            pl.BlockSpec((1, LD), lambda i: (0, 0)),
            pl.BlockSpec((LD, LD), lambda i: (0, 0)),
        ],
        out_specs=[
            pl.BlockSpec((EB, LD), lambda i: (i, 0)),
            pl.BlockSpec((2, EB, HALF), lambda i: (0, i, 0)),
        ],
        out_shape=[
            jax.ShapeDtypeStruct((N_EDGES, LD), jnp.bfloat16),
            jax.ShapeDtypeStruct((2, N_EDGES, HALF), jnp.float32),
        ],
    )(ef, g0, we, be, bc, wc)


def _pair_swap(x):
    up = jnp.roll(x, -1, axis=0)
    dn = jnp.roll(x, 1, axis=0)
    par = lax.broadcasted_iota(jnp.int32, x.shape, 0) % 2
    return jnp.where(par == 0, up, dn)


def _combine(g_ref, c_ref, m0b_ref):
    g = jnp.concatenate([g_ref[0], g_ref[1]], axis=1)
    c = jnp.concatenate([c_ref[0], c_ref[1]], axis=1)
    return jnp.maximum(g - _pair_swap(c) + m0b_ref[...].astype(jnp.float32),
                       0.0)


def _tc_level_body(g_ref, c_ref, m0b_ref, wc_ref, out_ref):
    x = _combine(g_ref, c_ref, m0b_ref)
    y = jnp.dot(x, wc_ref[...], preferred_element_type=jnp.float32)
    out_ref[0] = y[:, :HALF]
    out_ref[1] = y[:, HALF:]


def _tc_level(g, c, m0b, wc):
    return pl.pallas_call(
        _tc_level_body,
        grid=(EGRID,),
        in_specs=[
            pl.BlockSpec((2, EB, HALF), lambda i: (0, i, 0)),
            pl.BlockSpec((2, EB, HALF), lambda i: (0, i, 0)),
            pl.BlockSpec((EB, LD), lambda i: (i, 0)),
            pl.BlockSpec((LD, LD), lambda i: (0, 0)),
        ],
        out_specs=pl.BlockSpec((2, EB, HALF), lambda i: (0, i, 0)),
        out_shape=jax.ShapeDtypeStruct((2, N_EDGES, HALF), jnp.float32),
    )(g, c, m0b, wc)


def _tc_last_body(g_ref, c_ref, m0b_ref, out_ref):
    x = _combine(g_ref, c_ref, m0b_ref)
    out_ref[0] = x[:, :HALF]
    out_ref[1] = x[:, HALF:]


def _tc_last(g, c, m0b):
    return pl.pallas_call(
        _tc_last_body,
        grid=(EGRID,),
        in_specs=[
            pl.BlockSpec((2, EB, HALF), lambda i: (0, i, 0)),
            pl.BlockSpec((2, EB, HALF), lambda i: (0, i, 0)),
            pl.BlockSpec((EB, LD), lambda i: (i, 0)),
        ],
        out_specs=pl.BlockSpec((2, EB, HALF), lambda i: (0, i, 0)),
        out_shape=jax.ShapeDtypeStruct((2, N_EDGES, HALF), jnp.float32),
    )(g, c, m0b)


def _tc_out_body(e2n_ref, wo_ref, bo_ref, gid_ref, y_ref, acc_ref):
    i = pl.program_id(0)

    @pl.when(i == 0)
    def _():
        acc_ref[...] = jnp.zeros_like(acc_ref)

    h = jnp.maximum(jnp.concatenate([e2n_ref[0], e2n_ref[1]], axis=1), 0.0)
    o = jnp.dot(h, wo_ref[...], preferred_element_type=jnp.float32)
    o = jnp.maximum(o + bo_ref[...], 0.0)
    gid = gid_ref[0, 0, :]
    oh = (lax.broadcasted_iota(jnp.int32, (NG, NB), 0) == gid[None, :])
    acc_ref[...] += jnp.dot(oh.astype(jnp.float32), o,
                            preferred_element_type=jnp.float32)

    @pl.when(i == pl.num_programs(0) - 1)
    def _():
        y_ref[...] = jnp.maximum(acc_ref[...], 0.0)


def _tc_out(e2n, wo, bo, gid3):
    return pl.pallas_call(
        _tc_out_body,
        grid=(NGRID,),
        in_specs=[
            pl.BlockSpec((2, NB, HALF), lambda i: (0, i, 0)),
            pl.BlockSpec((LD, OD), lambda i: (0, 0)),
            pl.BlockSpec((1, OD), lambda i: (0, 0)),
            pl.BlockSpec((1, 1, NB), lambda i: (i, 0, 0)),
        ],
        out_specs=pl.BlockSpec((NG, OD), lambda i: (0, 0)),
        out_shape=jax.ShapeDtypeStruct((NG, OD), jnp.float32),
        scratch_shapes=[pltpu.VMEM((NG, OD), jnp.float32)],
    )(e2n, wo, bo, gid3)


NBUF = 3
NROUND = RPT // NBUF
TAIL = NROUND * NBUF


def _sc_gather_rows(table_ref, out_ref, c, s, idxbuf, rowbuf, gsem, ssem):
    def fire_gather(j, b):
        pltpu.async_copy(table_ref.at[idxbuf.at[j]], rowbuf.at[b], gsem)

    for b in range(NBUF):
        fire_gather(b, b)

    def rnd(r, carry):
        for b in range(NBUF):
            pltpu.make_async_copy(table_ref.at[idxbuf.at[r * NBUF + b]],
                                  rowbuf.at[b], gsem).wait()
        hs = [pltpu.async_copy(
                  rowbuf.at[b],
                  out_ref.at[c, pl.ds(s * EPT + (r * NBUF + b) * IDXW, IDXW)],
                  ssem) for b in range(NBUF)]
        for h in hs:
            h.wait()

        @pl.when(r + 1 < NROUND)
        def _():
            for b in range(NBUF):
                fire_gather((r + 1) * NBUF + b, b)

        return carry

    lax.fori_loop(0, NROUND, rnd, 0)
    for t in range(TAIL, RPT):
        b = t - TAIL
        pltpu.async_copy(table_ref.at[idxbuf.at[t]], rowbuf.at[b], gsem).wait()
        pltpu.sync_copy(rowbuf.at[b],
                        out_ref.at[c, pl.ds(s * EPT + t * IDXW, IDXW)])


_SC_SCRATCH = [
    pltpu.VMEM((RPT, IDXW), jnp.int32),
    pltpu.VMEM((NBUF, IDXW, HALF), jnp.float32),
    pltpu.SemaphoreType.DMA,
    pltpu.SemaphoreType.DMA,
]


@functools.partial(
    pl.kernel, mesh=_SC_MESH,
    out_type=jax.ShapeDtypeStruct((2, N_EDGES, HALF), jnp.float32),
    scratch_types=_SC_SCRATCH,
)
def _sc_gather(h0, h1, idx3, out, idxbuf, rowbuf, sem, sem2):
    c = lax.axis_index("c")
    s = lax.axis_index("s")
    pltpu.sync_copy(idx3.at[s], idxbuf)

    @pl.when(c == 0)
    def _():
        _sc_gather_rows(h0, out, c, s, idxbuf, rowbuf, sem, sem2)

    @pl.when(c == 1)
    def _():
        _sc_gather_rows(h1, out, c, s, idxbuf, rowbuf, sem, sem2)


def _sc_zero_acc(zer_ref, acc, s):
    @pl.when(s < NZTILES)
    def _():
        pltpu.sync_copy(zer_ref, acc.at[pl.ds(s * NZROWS, NZROWS)])


def _sc_scatter_add(cmat_ref, acc, c, s, idxbuf, rowbuf, lsem, asem):
    def src_slice(j):
        return cmat_ref.at[c, pl.ds(s * EPT + j * IDXW, IDXW)]

    for b in range(NBUF):
        pltpu.async_copy(src_slice(b), rowbuf.at[b], lsem)

    def rnd(r, carry):
        for b in range(NBUF):
            pltpu.make_async_copy(src_slice(r * NBUF + b), rowbuf.at[b],
                                  lsem).wait()
        hs = [pltpu.async_copy(rowbuf.at[b], acc.at[idxbuf.at[r * NBUF + b]],
                               asem, add=True) for b in range(NBUF)]
        for h in hs:
            h.wait()

        @pl.when(r + 1 < NROUND)
        def _():
            for b in range(NBUF):
                pltpu.async_copy(src_slice((r + 1) * NBUF + b), rowbuf.at[b],
                                 lsem)

        return carry

    lax.fori_loop(0, NROUND, rnd, 0)
    for t in range(TAIL, RPT):
        pltpu.sync_copy(src_slice(t), rowbuf.at[t - TAIL])
        pltpu.sync_copy(rowbuf.at[t - TAIL], acc.at[idxbuf.at[t]], add=True)


_LEVEL_SCRATCH = [
    pltpu.VMEM_SHARED((N_NODES, HALF), jnp.float32),
    pltpu.VMEM((RPT, IDXW), jnp.int32),
    pltpu.VMEM((NBUF, IDXW, HALF), jnp.float32),
    pltpu.SemaphoreType.DMA,
    pltpu.SemaphoreType.DMA,
]


@functools.partial(
    pl.kernel, mesh=_SC_MESH,
    out_type=jax.ShapeDtypeStruct((2, N_EDGES, HALF), jnp.float32),
    scratch_types=_LEVEL_SCRATCH,
)
def _sc_level(cmat, src3, dst3, zer, g_out, acc, idxbuf, rowbuf, sem, sem2):
    c = lax.axis_index("c")
    s = lax.axis_index("s")
    pltpu.sync_copy(dst3.at[s], idxbuf)
    _sc_zero_acc(zer, acc, s)
    plsc.subcore_barrier()
    _sc_scatter_add(cmat, acc, c, s, idxbuf, rowbuf, sem, sem2)
    plsc.subcore_barrier()
    pltpu.sync_copy(src3.at[s], idxbuf)
    _sc_gather_rows(acc, g_out, c, s, idxbuf, rowbuf, sem, sem2)


@functools.partial(
    pl.kernel, mesh=_SC_MESH,
    out_type=jax.ShapeDtypeStruct((2, N_NODES, HALF), jnp.float32),
    scratch_types=_LEVEL_SCRATCH,
)
def _sc_scatter(cmat, dst3, zer, out, acc, idxbuf, rowbuf, sem, sem2):
    c = lax.axis_index("c")
    s = lax.axis_index("s")
    pltpu.sync_copy(dst3.at[s], idxbuf)
    _sc_zero_acc(zer, acc, s)
    plsc.subcore_barrier()
    _sc_scatter_add(cmat, acc, c, s, idxbuf, rowbuf, sem, sem2)
    plsc.subcore_barrier()

    @pl.when(s < NZTILES)
    def _():
        pltpu.sync_copy(acc.at[pl.ds(s * NZROWS, NZROWS)],
                        out.at[c, pl.ds(s * NZROWS, NZROWS)])


def kernel(node_feat, edge_feat, edge_index, graph_ids, W_n2l, b_n2l,
           W_e2l, b_e2l, W_conv, b_conv, W_out, b_out):
    src3 = edge_index[0].reshape(TILES, RPT, IDXW)
    dst3 = edge_index[1].reshape(TILES, RPT, IDXW)
    gid3 = graph_ids.reshape(NGRID, 1, NB)
    zer = jnp.zeros((NZROWS, HALF), jnp.float32)
    bn = b_n2l.reshape(1, LD)
    be = b_e2l.reshape(1, LD)
    bc = b_conv.reshape(1, LD)
    bo = b_out.reshape(1, OD)

    h = _tc_node_linear(node_feat, W_n2l, bn)
    g0 = _sc_gather(h[0], h[1], src3)
    m0b, c = _tc_input_linear(edge_feat, g0, W_e2l, be, bc, W_conv)
    for _ in range(2):
        g = _sc_level(c, src3, dst3, zer)
        c = _tc_level(g, c, m0b, W_conv)
    g = _sc_level(c, src3, dst3, zer)
    cur = _tc_last(g, c, m0b)
    e2n = _sc_scatter(cur, dst3, zer)
    return _tc_out(e2n, W_out, bo, gid3)


# speedup vs baseline: 2.7036x; 1.0301x over previous
import functools

import jax
import jax.numpy as jnp
from jax import lax
from jax.experimental import pallas as pl
from jax.experimental.pallas import tpu as pltpu
from jax.experimental.pallas import tpu_sc as plsc

N_NODES = 10000
N_EDGES = 160000
DF = 256
DE = 16
LD = 256
OD = 128
NG = 128
HALF = 128

EB = 1000
NB = 1000
EGRID = N_EDGES // EB
NGRID = N_NODES // NB

TILES = 16
EPT = N_EDGES // TILES
IDXW = 80
RPT = EPT // IDXW
NZROWS = 1000
NZTILES = N_NODES // NZROWS

_SC_MESH = plsc.VectorSubcoreMesh(core_axis_name="c", subcore_axis_name="s")


def _tc_node_linear_body(nf_ref, w_ref, b_ref, h_ref):
    h = jnp.dot(nf_ref[...], w_ref[...], preferred_element_type=jnp.float32)
    h = h + b_ref[...]
    h_ref[0] = h[:, :HALF]
    h_ref[1] = h[:, HALF:]


def _tc_node_linear(nf, w, b):
    return pl.pallas_call(
        _tc_node_linear_body,
        grid=(NGRID,),
        in_specs=[
            pl.BlockSpec((NB, DF), lambda i: (i, 0)),
            pl.BlockSpec((DF, LD), lambda i: (0, 0)),
            pl.BlockSpec((1, LD), lambda i: (0, 0)),
        ],
        out_specs=pl.BlockSpec((2, NB, HALF), lambda i: (0, i, 0)),
        out_shape=jax.ShapeDtypeStruct((2, N_NODES, HALF), jnp.float32),
    )(nf, w, b)


def _tc_input_linear_body(ef_ref, g_ref, we_ref, be_ref, bc_ref, wc_ref,
                          m0b_ref, c1_ref):
    g = jnp.concatenate([g_ref[0], g_ref[1]], axis=1)
    m0 = jnp.dot(ef_ref[...], we_ref[...], preferred_element_type=jnp.float32)
    m0 = m0 + be_ref[...] + g
    m0b_ref[...] = (m0 + bc_ref[...]).astype(jnp.bfloat16)
    c1 = jnp.dot(jnp.maximum(m0, 0.0), wc_ref[...],
                 preferred_element_type=jnp.float32)
    c1_ref[0] = c1[:, :HALF]
    c1_ref[1] = c1[:, HALF:]


def _tc_input_linear(ef, g0, we, be, bc, wc):
    return pl.pallas_call(
        _tc_input_linear_body,
        grid=(EGRID,),
        in_specs=[
            pl.BlockSpec((EB, DE), lambda i: (i, 0)),
            pl.BlockSpec((2, EB, HALF), lambda i: (0, i, 0)),
            pl.BlockSpec((DE, LD), lambda i: (0, 0)),
            pl.BlockSpec((1, LD), lambda i: (0, 0)),
            pl.BlockSpec((1, LD), lambda i: (0, 0)),
            pl.BlockSpec((LD, LD), lambda i: (0, 0)),
        ],
        out_specs=[
            pl.BlockSpec((EB, LD), lambda i: (i, 0)),
            pl.BlockSpec((2, EB, HALF), lambda i: (0, i, 0)),
        ],
        out_shape=[
            jax.ShapeDtypeStruct((N_EDGES, LD), jnp.bfloat16),
            jax.ShapeDtypeStruct((2, N_EDGES, HALF), jnp.float32),
        ],
    )(ef, g0, we, be, bc, wc)


def _pair_swap(x):
    up = jnp.roll(x, -1, axis=0)
    dn = jnp.roll(x, 1, axis=0)
    par = lax.broadcasted_iota(jnp.int32, x.shape, 0) % 2
    return jnp.where(par == 0, up, dn)


def _combine(g_ref, c_ref, m0b_ref):
    g = jnp.concatenate([g_ref[0], g_ref[1]], axis=1)
    c = jnp.concatenate([c_ref[0], c_ref[1]], axis=1)
    return jnp.maximum(g - _pair_swap(c) + m0b_ref[...].astype(jnp.float32),
                       0.0)


def _tc_level_body(g_ref, c_ref, m0b_ref, wc_ref, out_ref):
    x = _combine(g_ref, c_ref, m0b_ref)
    y = jnp.dot(x, wc_ref[...], preferred_element_type=jnp.float32)
    out_ref[0] = y[:, :HALF]
    out_ref[1] = y[:, HALF:]


def _tc_level(g, c, m0b, wc):
    return pl.pallas_call(
        _tc_level_body,
        grid=(EGRID,),
        in_specs=[
            pl.BlockSpec((2, EB, HALF), lambda i: (0, i, 0)),
            pl.BlockSpec((2, EB, HALF), lambda i: (0, i, 0)),
            pl.BlockSpec((EB, LD), lambda i: (i, 0)),
            pl.BlockSpec((LD, LD), lambda i: (0, 0)),
        ],
        out_specs=pl.BlockSpec((2, EB, HALF), lambda i: (0, i, 0)),
        out_shape=jax.ShapeDtypeStruct((2, N_EDGES, HALF), jnp.float32),
    )(g, c, m0b, wc)


def _tc_last_body(g_ref, c_ref, m0b_ref, out_ref):
    x = _combine(g_ref, c_ref, m0b_ref)
    out_ref[0] = x[:, :HALF]
    out_ref[1] = x[:, HALF:]


def _tc_last(g, c, m0b):
    return pl.pallas_call(
        _tc_last_body,
        grid=(EGRID,),
        in_specs=[
            pl.BlockSpec((2, EB, HALF), lambda i: (0, i, 0)),
            pl.BlockSpec((2, EB, HALF), lambda i: (0, i, 0)),
            pl.BlockSpec((EB, LD), lambda i: (i, 0)),
        ],
        out_specs=pl.BlockSpec((2, EB, HALF), lambda i: (0, i, 0)),
        out_shape=jax.ShapeDtypeStruct((2, N_EDGES, HALF), jnp.float32),
    )(g, c, m0b)


def _tc_out_body(e2n_ref, wo_ref, bo_ref, gid_ref, y_ref, acc_ref):
    i = pl.program_id(0)

    @pl.when(i == 0)
    def _():
        acc_ref[...] = jnp.zeros_like(acc_ref)

    h = jnp.maximum(jnp.concatenate([e2n_ref[0], e2n_ref[1]], axis=1), 0.0)
    o = jnp.dot(h, wo_ref[...], preferred_element_type=jnp.float32)
    o = jnp.maximum(o + bo_ref[...], 0.0)
    gid = gid_ref[0, 0, :]
    oh = (lax.broadcasted_iota(jnp.int32, (NG, NB), 0) == gid[None, :])
    acc_ref[...] += jnp.dot(oh.astype(jnp.float32), o,
                            preferred_element_type=jnp.float32)

    @pl.when(i == pl.num_programs(0) - 1)
    def _():
        y_ref[...] = jnp.maximum(acc_ref[...], 0.0)


def _tc_out(e2n, wo, bo, gid3):
    return pl.pallas_call(
        _tc_out_body,
        grid=(NGRID,),
        in_specs=[
            pl.BlockSpec((2, NB, HALF), lambda i: (0, i, 0)),
            pl.BlockSpec((LD, OD), lambda i: (0, 0)),
            pl.BlockSpec((1, OD), lambda i: (0, 0)),
            pl.BlockSpec((1, 1, NB), lambda i: (i, 0, 0)),
        ],
        out_specs=pl.BlockSpec((NG, OD), lambda i: (0, 0)),
        out_shape=jax.ShapeDtypeStruct((NG, OD), jnp.float32),
        scratch_shapes=[pltpu.VMEM((NG, OD), jnp.float32)],
    )(e2n, wo, bo, gid3)


NBUF = 3
NROUND = RPT // NBUF
TAIL = NROUND * NBUF


def _sc_gather_rows(table_ref, out_ref, c, s, idxbuf, rowbuf, gsem, ssem):
    def fire_gather(j, b):
        pltpu.async_copy(table_ref.at[idxbuf.at[j]], rowbuf.at[b], gsem)

    for b in range(NBUF):
        fire_gather(b, b)

    def rnd(r, carry):
        for b in range(NBUF):
            pltpu.make_async_copy(table_ref.at[idxbuf.at[r * NBUF + b]],
                                  rowbuf.at[b], gsem).wait()
        hs = [pltpu.async_copy(
                  rowbuf.at[b],
                  out_ref.at[c, pl.ds(s * EPT + (r * NBUF + b) * IDXW, IDXW)],
                  ssem) for b in range(NBUF)]
        for h in hs:
            h.wait()

        @pl.when(r + 1 < NROUND)
        def _():
            for b in range(NBUF):
                fire_gather((r + 1) * NBUF + b, b)

        return carry

    lax.fori_loop(0, NROUND, rnd, 0)
    for t in range(TAIL, RPT):
        b = t - TAIL
        pltpu.async_copy(table_ref.at[idxbuf.at[t]], rowbuf.at[b], gsem).wait()
        pltpu.sync_copy(rowbuf.at[b],
                        out_ref.at[c, pl.ds(s * EPT + t * IDXW, IDXW)])


_SC_SCRATCH = [
    pltpu.VMEM((RPT, IDXW), jnp.int32),
    pltpu.VMEM((NBUF, IDXW, HALF), jnp.float32),
    pltpu.SemaphoreType.DMA,
    pltpu.SemaphoreType.DMA,
]


def _load_table(tab_ref, acc, c, s):
    @pl.when(s < NZTILES)
    def _():
        pltpu.sync_copy(tab_ref.at[c, pl.ds(s * NZROWS, NZROWS)],
                        acc.at[pl.ds(s * NZROWS, NZROWS)])


def _sc_zero_acc(zer_ref, acc, s):
    @pl.when(s < NZTILES)
    def _():
        pltpu.sync_copy(zer_ref, acc.at[pl.ds(s * NZROWS, NZROWS)])


def _sc_scatter_add(cmat_ref, acc, c, s, idxbuf, rowbuf, lsem, asem):
    def src_slice(j):
        return cmat_ref.at[c, pl.ds(s * EPT + j * IDXW, IDXW)]

    for b in range(NBUF):
        pltpu.async_copy(src_slice(b), rowbuf.at[b], lsem)

    def rnd(r, carry):
        for b in range(NBUF):
            pltpu.make_async_copy(src_slice(r * NBUF + b), rowbuf.at[b],
                                  lsem).wait()
        hs = [pltpu.async_copy(rowbuf.at[b], acc.at[idxbuf.at[r * NBUF + b]],
                               asem, add=True) for b in range(NBUF)]
        for h in hs:
            h.wait()

        @pl.when(r + 1 < NROUND)
        def _():
            for b in range(NBUF):
                pltpu.async_copy(src_slice((r + 1) * NBUF + b), rowbuf.at[b],
                                 lsem)

        return carry

    lax.fori_loop(0, NROUND, rnd, 0)
    for t in range(TAIL, RPT):
        pltpu.sync_copy(src_slice(t), rowbuf.at[t - TAIL])
        pltpu.sync_copy(rowbuf.at[t - TAIL], acc.at[idxbuf.at[t]], add=True)


_LEVEL_SCRATCH = [
    pltpu.VMEM_SHARED((N_NODES, HALF), jnp.float32),
    pltpu.VMEM((RPT, IDXW), jnp.int32),
    pltpu.VMEM((NBUF, IDXW, HALF), jnp.float32),
    pltpu.SemaphoreType.DMA,
    pltpu.SemaphoreType.DMA,
]


@functools.partial(
    pl.kernel, mesh=_SC_MESH,
    out_type=jax.ShapeDtypeStruct((2, N_EDGES, HALF), jnp.float32),
    scratch_types=_LEVEL_SCRATCH,
)
def _sc_gather(h, idx3, out, acc, idxbuf, rowbuf, sem, sem2):
    c = lax.axis_index("c")
    s = lax.axis_index("s")
    pltpu.sync_copy(idx3.at[s], idxbuf)
    _load_table(h, acc, c, s)
    plsc.subcore_barrier()
    _sc_gather_rows(acc, out, c, s, idxbuf, rowbuf, sem, sem2)


@functools.partial(
    pl.kernel, mesh=_SC_MESH,
    out_type=jax.ShapeDtypeStruct((2, N_EDGES, HALF), jnp.float32),
    scratch_types=_LEVEL_SCRATCH,
)
def _sc_level(cmat, src3, dst3, zer, g_out, acc, idxbuf, rowbuf, sem, sem2):
    c = lax.axis_index("c")
    s = lax.axis_index("s")
    pltpu.sync_copy(dst3.at[s], idxbuf)
    _sc_zero_acc(zer, acc, s)
    plsc.subcore_barrier()
    _sc_scatter_add(cmat, acc, c, s, idxbuf, rowbuf, sem, sem2)
    plsc.subcore_barrier()
    pltpu.sync_copy(src3.at[s], idxbuf)
    _sc_gather_rows(acc, g_out, c, s, idxbuf, rowbuf, sem, sem2)


@functools.partial(
    pl.kernel, mesh=_SC_MESH,
    out_type=jax.ShapeDtypeStruct((2, N_NODES, HALF), jnp.float32),
    scratch_types=_LEVEL_SCRATCH,
)
def _sc_scatter(cmat, dst3, zer, out, acc, idxbuf, rowbuf, sem, sem2):
    c = lax.axis_index("c")
    s = lax.axis_index("s")
    pltpu.sync_copy(dst3.at[s], idxbuf)
    _sc_zero_acc(zer, acc, s)
    plsc.subcore_barrier()
    _sc_scatter_add(cmat, acc, c, s, idxbuf, rowbuf, sem, sem2)
    plsc.subcore_barrier()

    @pl.when(s < NZTILES)
    def _():
        pltpu.sync_copy(acc.at[pl.ds(s * NZROWS, NZROWS)],
                        out.at[c, pl.ds(s * NZROWS, NZROWS)])


def kernel(node_feat, edge_feat, edge_index, graph_ids, W_n2l, b_n2l,
           W_e2l, b_e2l, W_conv, b_conv, W_out, b_out):
    src3 = edge_index[0].reshape(TILES, RPT, IDXW)
    dst3 = edge_index[1].reshape(TILES, RPT, IDXW)
    gid3 = graph_ids.reshape(NGRID, 1, NB)
    zer = jnp.zeros((NZROWS, HALF), jnp.float32)
    bn = b_n2l.reshape(1, LD)
    be = b_e2l.reshape(1, LD)
    bc = b_conv.reshape(1, LD)
    bo = b_out.reshape(1, OD)

    h = _tc_node_linear(node_feat, W_n2l, bn)
    g0 = _sc_gather(h, src3)
    m0b, c = _tc_input_linear(edge_feat, g0, W_e2l, be, bc, W_conv)
    for _ in range(2):
        g = _sc_level(c, src3, dst3, zer)
        c = _tc_level(g, c, m0b, W_conv)
    g = _sc_level(c, src3, dst3, zer)
    cur = _tc_last(g, c, m0b)
    e2n = _sc_scatter(cur, dst3, zer)
    return _tc_out(e2n, W_out, bo, gid3)


# speedup vs baseline: 3.1120x; 1.1511x over previous
import functools

import jax
import jax.numpy as jnp
from jax import lax
from jax.experimental import pallas as pl
from jax.experimental.pallas import tpu as pltpu
from jax.experimental.pallas import tpu_sc as plsc

N_NODES = 10000
N_EDGES = 160000
DF = 256
DE = 16
LD = 256
OD = 128
NG = 128
HALF = 128

EB = 1000
NB = 1000
EGRID = N_EDGES // EB
NGRID = N_NODES // NB

TILES = 16
EPT = N_EDGES // TILES
IDXW = 80
RPT = EPT // IDXW
NZROWS = 1000
NZTILES = N_NODES // NZROWS

_SC_MESH = plsc.VectorSubcoreMesh(core_axis_name="c", subcore_axis_name="s")


def _tc_node_linear_body(nf_ref, w_ref, b_ref, h_ref):
    h = jnp.dot(nf_ref[...], w_ref[...], preferred_element_type=jnp.float32)
    h = h + b_ref[...]
    h_ref[0] = h[:, :HALF]
    h_ref[1] = h[:, HALF:]


def _tc_node_linear(nf, w, b):
    return pl.pallas_call(
        _tc_node_linear_body,
        grid=(NGRID,),
        in_specs=[
            pl.BlockSpec((NB, DF), lambda i: (i, 0)),
            pl.BlockSpec((DF, LD), lambda i: (0, 0)),
            pl.BlockSpec((1, LD), lambda i: (0, 0)),
        ],
        out_specs=pl.BlockSpec((2, NB, HALF), lambda i: (0, i, 0)),
        out_shape=jax.ShapeDtypeStruct((2, N_NODES, HALF), jnp.float32),
    )(nf, w, b)


def _tc_input_linear_body(ef_ref, g_ref, we_ref, be_ref, bc_ref, wc_ref,
                          m0b_ref, c1_ref):
    g = jnp.concatenate([g_ref[0], g_ref[1]], axis=1)
    m0 = jnp.dot(ef_ref[...], we_ref[...], preferred_element_type=jnp.float32)
    m0 = m0 + be_ref[...] + g
    m0b_ref[...] = (m0 + bc_ref[...]).astype(jnp.bfloat16)
    c1 = jnp.dot(jnp.maximum(m0, 0.0).astype(jnp.bfloat16), wc_ref[...],
                 preferred_element_type=jnp.float32)
    c1_ref[0] = c1[:, :HALF]
    c1_ref[1] = c1[:, HALF:]


def _tc_input_linear(ef, g0, we, be, bc, wc):
    return pl.pallas_call(
        _tc_input_linear_body,
        grid=(EGRID,),
        in_specs=[
            pl.BlockSpec((EB, DE), lambda i: (i, 0)),
            pl.BlockSpec((2, EB, HALF), lambda i: (0, i, 0)),
            pl.BlockSpec((DE, LD), lambda i: (0, 0)),
            pl.BlockSpec((1, LD), lambda i: (0, 0)),
            pl.BlockSpec((1, LD), lambda i: (0, 0)),
            pl.BlockSpec((LD, LD), lambda i: (0, 0)),
        ],
        out_specs=[
            pl.BlockSpec((EB, LD), lambda i: (i, 0)),
            pl.BlockSpec((2, EB, HALF), lambda i: (0, i, 0)),
        ],
        out_shape=[
            jax.ShapeDtypeStruct((N_EDGES, LD), jnp.bfloat16),
            jax.ShapeDtypeStruct((2, N_EDGES, HALF), jnp.float32),
        ],
    )(ef, g0, we, be, bc, wc)


def _pair_swap(x):
    up = jnp.roll(x, -1, axis=0)
    dn = jnp.roll(x, 1, axis=0)
    par = lax.broadcasted_iota(jnp.int32, x.shape, 0) % 2
    return jnp.where(par == 0, up, dn)


def _combine(g_ref, c_ref, m0b_ref):
    g = jnp.concatenate([g_ref[0], g_ref[1]], axis=1)
    c = jnp.concatenate([c_ref[0], c_ref[1]], axis=1)
    return jnp.maximum(g - _pair_swap(c) + m0b_ref[...].astype(jnp.float32),
                       0.0)


def _tc_level_body(g_ref, c_ref, m0b_ref, wc_ref, out_ref):
    x = _combine(g_ref, c_ref, m0b_ref)
    y = jnp.dot(x.astype(jnp.bfloat16), wc_ref[...],
                preferred_element_type=jnp.float32)
    out_ref[0] = y[:, :HALF]
    out_ref[1] = y[:, HALF:]


def _tc_level(g, c, m0b, wc):
    return pl.pallas_call(
        _tc_level_body,
        grid=(EGRID,),
        in_specs=[
            pl.BlockSpec((2, EB, HALF), lambda i: (0, i, 0)),
            pl.BlockSpec((2, EB, HALF), lambda i: (0, i, 0)),
            pl.BlockSpec((EB, LD), lambda i: (i, 0)),
            pl.BlockSpec((LD, LD), lambda i: (0, 0)),
        ],
        out_specs=pl.BlockSpec((2, EB, HALF), lambda i: (0, i, 0)),
        out_shape=jax.ShapeDtypeStruct((2, N_EDGES, HALF), jnp.float32),
    )(g, c, m0b, wc)


def _tc_last_body(g_ref, c_ref, m0b_ref, out_ref):
    x = _combine(g_ref, c_ref, m0b_ref)
    out_ref[0] = x[:, :HALF]
    out_ref[1] = x[:, HALF:]


def _tc_last(g, c, m0b):
    return pl.pallas_call(
        _tc_last_body,
        grid=(EGRID,),
        in_specs=[
            pl.BlockSpec((2, EB, HALF), lambda i: (0, i, 0)),
            pl.BlockSpec((2, EB, HALF), lambda i: (0, i, 0)),
            pl.BlockSpec((EB, LD), lambda i: (i, 0)),
        ],
        out_specs=pl.BlockSpec((2, EB, HALF), lambda i: (0, i, 0)),
        out_shape=jax.ShapeDtypeStruct((2, N_EDGES, HALF), jnp.float32),
    )(g, c, m0b)


def _tc_out_body(e2n_ref, wo_ref, bo_ref, gid_ref, y_ref, acc_ref):
    i = pl.program_id(0)

    @pl.when(i == 0)
    def _():
        acc_ref[...] = jnp.zeros_like(acc_ref)

    h = jnp.maximum(jnp.concatenate([e2n_ref[0], e2n_ref[1]], axis=1), 0.0)
    o = jnp.dot(h, wo_ref[...], preferred_element_type=jnp.float32)
    o = jnp.maximum(o + bo_ref[...], 0.0)
    gid = gid_ref[0, 0, :]
    oh = (lax.broadcasted_iota(jnp.int32, (NG, NB), 0) == gid[None, :])
    acc_ref[...] += jnp.dot(oh.astype(jnp.float32), o,
                            preferred_element_type=jnp.float32)

    @pl.when(i == pl.num_programs(0) - 1)
    def _():
        y_ref[...] = jnp.maximum(acc_ref[...], 0.0)


def _tc_out(e2n, wo, bo, gid3):
    return pl.pallas_call(
        _tc_out_body,
        grid=(NGRID,),
        in_specs=[
            pl.BlockSpec((2, NB, HALF), lambda i: (0, i, 0)),
            pl.BlockSpec((LD, OD), lambda i: (0, 0)),
            pl.BlockSpec((1, OD), lambda i: (0, 0)),
            pl.BlockSpec((1, 1, NB), lambda i: (i, 0, 0)),
        ],
        out_specs=pl.BlockSpec((NG, OD), lambda i: (0, 0)),
        out_shape=jax.ShapeDtypeStruct((NG, OD), jnp.float32),
        scratch_shapes=[pltpu.VMEM((NG, OD), jnp.float32)],
    )(e2n, wo, bo, gid3)


NBUF = 3
NROUND = RPT // NBUF
TAIL = NROUND * NBUF


def _sc_gather_rows(table_ref, out_ref, c, s, idxbuf, rowbuf, gsem, ssem):
    def fire_gather(j, b):
        pltpu.async_copy(table_ref.at[idxbuf.at[j]], rowbuf.at[b], gsem)

    for b in range(NBUF):
        fire_gather(b, b)

    def rnd(r, carry):
        for b in range(NBUF):
            pltpu.make_async_copy(table_ref.at[idxbuf.at[r * NBUF + b]],
                                  rowbuf.at[b], gsem).wait()
        hs = [pltpu.async_copy(
                  rowbuf.at[b],
                  out_ref.at[c, pl.ds(s * EPT + (r * NBUF + b) * IDXW, IDXW)],
                  ssem) for b in range(NBUF)]
        for h in hs:
            h.wait()

        @pl.when(r + 1 < NROUND)
        def _():
            for b in range(NBUF):
                fire_gather((r + 1) * NBUF + b, b)

        return carry

    lax.fori_loop(0, NROUND, rnd, 0)
    for t in range(TAIL, RPT):
        b = t - TAIL
        pltpu.async_copy(table_ref.at[idxbuf.at[t]], rowbuf.at[b], gsem).wait()
        pltpu.sync_copy(rowbuf.at[b],
                        out_ref.at[c, pl.ds(s * EPT + t * IDXW, IDXW)])


_SC_SCRATCH = [
    pltpu.VMEM((RPT, IDXW), jnp.int32),
    pltpu.VMEM((NBUF, IDXW, HALF), jnp.float32),
    pltpu.SemaphoreType.DMA,
    pltpu.SemaphoreType.DMA,
]


def _load_table(tab_ref, acc, c, s):
    @pl.when(s < NZTILES)
    def _():
        pltpu.sync_copy(tab_ref.at[c, pl.ds(s * NZROWS, NZROWS)],
                        acc.at[pl.ds(s * NZROWS, NZROWS)])


def _sc_zero_acc(zer_ref, acc, s):
    @pl.when(s < NZTILES)
    def _():
        pltpu.sync_copy(zer_ref, acc.at[pl.ds(s * NZROWS, NZROWS)])


def _sc_scatter_add(cmat_ref, acc, c, s, idxbuf, rowbuf, lsem, asem):
    def src_slice(j):
        return cmat_ref.at[c, pl.ds(s * EPT + j * IDXW, IDXW)]

    for b in range(NBUF):
        pltpu.async_copy(src_slice(b), rowbuf.at[b], lsem)

    def rnd(r, carry):
        for b in range(NBUF):
            pltpu.make_async_copy(src_slice(r * NBUF + b), rowbuf.at[b],
                                  lsem).wait()
        hs = [pltpu.async_copy(rowbuf.at[b], acc.at[idxbuf.at[r * NBUF + b]],
                               asem, add=True) for b in range(NBUF)]
        for h in hs:
            h.wait()

        @pl.when(r + 1 < NROUND)
        def _():
            for b in range(NBUF):
                pltpu.async_copy(src_slice((r + 1) * NBUF + b), rowbuf.at[b],
                                 lsem)

        return carry

    lax.fori_loop(0, NROUND, rnd, 0)
    for t in range(TAIL, RPT):
        pltpu.sync_copy(src_slice(t), rowbuf.at[t - TAIL])
        pltpu.sync_copy(rowbuf.at[t - TAIL], acc.at[idxbuf.at[t]], add=True)


_LEVEL_SCRATCH = [
    pltpu.VMEM_SHARED((N_NODES, HALF), jnp.float32),
    pltpu.VMEM((RPT, IDXW), jnp.int32),
    pltpu.VMEM((NBUF, IDXW, HALF), jnp.float32),
    pltpu.SemaphoreType.DMA,
    pltpu.SemaphoreType.DMA,
]


@functools.partial(
    pl.kernel, mesh=_SC_MESH,
    out_type=jax.ShapeDtypeStruct((2, N_EDGES, HALF), jnp.float32),
    scratch_types=_LEVEL_SCRATCH,
)
def _sc_gather(h, idx3, out, acc, idxbuf, rowbuf, sem, sem2):
    c = lax.axis_index("c")
    s = lax.axis_index("s")
    pltpu.sync_copy(idx3.at[s], idxbuf)
    _load_table(h, acc, c, s)
    plsc.subcore_barrier()
    _sc_gather_rows(acc, out, c, s, idxbuf, rowbuf, sem, sem2)


@functools.partial(
    pl.kernel, mesh=_SC_MESH,
    out_type=jax.ShapeDtypeStruct((2, N_EDGES, HALF), jnp.float32),
    scratch_types=_LEVEL_SCRATCH,
)
def _sc_level(cmat, src3, dst3, zer, g_out, acc, idxbuf, rowbuf, sem, sem2):
    c = lax.axis_index("c")
    s = lax.axis_index("s")
    pltpu.sync_copy(dst3.at[s], idxbuf)
    _sc_zero_acc(zer, acc, s)
    plsc.subcore_barrier()
    _sc_scatter_add(cmat, acc, c, s, idxbuf, rowbuf, sem, sem2)
    plsc.subcore_barrier()
    pltpu.sync_copy(src3.at[s], idxbuf)
    _sc_gather_rows(acc, g_out, c, s, idxbuf, rowbuf, sem, sem2)


@functools.partial(
    pl.kernel, mesh=_SC_MESH,
    out_type=jax.ShapeDtypeStruct((2, N_NODES, HALF), jnp.float32),
    scratch_types=_LEVEL_SCRATCH,
)
def _sc_scatter(cmat, dst3, zer, out, acc, idxbuf, rowbuf, sem, sem2):
    c = lax.axis_index("c")
    s = lax.axis_index("s")
    pltpu.sync_copy(dst3.at[s], idxbuf)
    _sc_zero_acc(zer, acc, s)
    plsc.subcore_barrier()
    _sc_scatter_add(cmat, acc, c, s, idxbuf, rowbuf, sem, sem2)
    plsc.subcore_barrier()

    @pl.when(s < NZTILES)
    def _():
        pltpu.sync_copy(acc.at[pl.ds(s * NZROWS, NZROWS)],
                        out.at[c, pl.ds(s * NZROWS, NZROWS)])


def kernel(node_feat, edge_feat, edge_index, graph_ids, W_n2l, b_n2l,
           W_e2l, b_e2l, W_conv, b_conv, W_out, b_out):
    src3 = edge_index[0].reshape(TILES, RPT, IDXW)
    dst3 = edge_index[1].reshape(TILES, RPT, IDXW)
    gid3 = graph_ids.reshape(NGRID, 1, NB)
    zer = jnp.zeros((NZROWS, HALF), jnp.float32)
    wc16 = W_conv.astype(jnp.bfloat16)
    bn = b_n2l.reshape(1, LD)
    be = b_e2l.reshape(1, LD)
    bc = b_conv.reshape(1, LD)
    bo = b_out.reshape(1, OD)

    h = _tc_node_linear(node_feat, W_n2l, bn)
    g0 = _sc_gather(h, src3)
    m0b, c = _tc_input_linear(edge_feat, g0, W_e2l, be, bc, wc16)
    for _ in range(2):
        g = _sc_level(c, src3, dst3, zer)
        c = _tc_level(g, c, m0b, wc16)
    g = _sc_level(c, src3, dst3, zer)
    cur = _tc_last(g, c, m0b)
    e2n = _sc_scatter(cur, dst3, zer)
    return _tc_out(e2n, W_out, bo, gid3)
